# Initial kernel scaffold; baseline (speedup 1.0000x reference)
import jax
import jax.numpy as jnp
from jax.experimental import pallas as pl


def kernel(X, w_embed):
    raise NotImplementedError("write your pallas kernel here")


# speedup vs baseline: 5.8933x; 5.8933x over previous
import functools

import jax
import jax.numpy as jnp
from jax import lax
from jax.experimental import pallas as pl
from jax.experimental.pallas import tpu as pltpu
from jax.experimental.pallas import tpu_sc as plsc

B, T, D = 1024, 200, 64
N = B * T
NC, NS, L = 2, 16, 16
NW = NC * NS
PER_W = N // NW
CH = 128
G = PER_W // CH

_mesh = plsc.VectorSubcoreMesh(core_axis_name="c", subcore_axis_name="s")


@functools.partial(
    pl.kernel,
    mesh=_mesh,
    out_type=jax.ShapeDtypeStruct((N, D), jnp.float32),
    scratch_types=[
        pltpu.VMEM((1, G, CH), jnp.int32),
        pltpu.VMEM((1, G, CH), jnp.int32),
        pltpu.VMEM((CH, D), jnp.float32),
        pltpu.VMEM((CH, D), jnp.float32),
        pltpu.SemaphoreType.DMA,
        pltpu.SemaphoreType.DMA,
    ],
    compiler_params=pltpu.CompilerParams(use_tc_tiling_on_sc=False),
)
def _sc_embed(idx0_hbm, idx1_hbm, tab_hbm, out_hbm,
              idx0_v, idx1_v, buf0, buf1, g0, g1):
    wid = lax.axis_index("s") * NC + lax.axis_index("c")
    gbase = wid * G
    pltpu.sync_copy(idx0_hbm.at[pl.ds(wid, 1)], idx0_v)
    pltpu.sync_copy(idx1_hbm.at[pl.ds(wid, 1)], idx1_v)

    def step(j, carry):
        c0 = pltpu.async_copy(tab_hbm.at[idx0_v.at[0, j]], buf0, g0)
        c1 = pltpu.async_copy(tab_hbm.at[idx1_v.at[0, j]], buf1, g1)
        c0.wait()
        c1.wait()

        def addrow(r, rc):
            for c in range(D // L):
                sl = pl.ds(c * L, L)
                plsc.addupdate(buf0.at[r, sl], buf1[r, sl])
            return rc

        lax.fori_loop(0, CH, addrow, 0, unroll=2)
        pltpu.sync_copy(buf0, out_hbm.at[pl.ds((gbase + j) * CH, CH)])
        return carry

    lax.fori_loop(0, G, step, 0)


def kernel(X, w_embed):
    Xf = X.reshape(N, 2).astype(jnp.int32)
    idx0 = Xf[:, 0].reshape(NW, G, CH)
    idx1 = Xf[:, 1].reshape(NW, G, CH)
    h = _sc_embed(idx0, idx1, w_embed)
    return h.reshape(B, T, D), w_embed


# speedup vs baseline: 6.4166x; 1.0888x over previous
import functools

import jax
import jax.numpy as jnp
from jax import lax
from jax.experimental import pallas as pl
from jax.experimental.pallas import tpu as pltpu
from jax.experimental.pallas import tpu_sc as plsc

B, T, D = 1024, 200, 64
N = B * T
NC, NS, L = 2, 16, 16
NW = NC * NS
PER_W = N // NW
CH = 128
G = PER_W // CH

_mesh = plsc.VectorSubcoreMesh(core_axis_name="c", subcore_axis_name="s")


@functools.partial(
    pl.kernel,
    mesh=_mesh,
    out_type=jax.ShapeDtypeStruct((N, D), jnp.float32),
    scratch_types=[
        pltpu.VMEM((1, G, CH), jnp.int32),
        pltpu.VMEM((1, G, CH), jnp.int32),
        pltpu.VMEM((2, CH, D), jnp.float32),
        pltpu.SemaphoreType.DMA,
        pltpu.SemaphoreType.DMA,
        pltpu.SemaphoreType.DMA,
        pltpu.SemaphoreType.DMA,
    ],
    compiler_params=pltpu.CompilerParams(use_tc_tiling_on_sc=False),
)
def _sc_embed(idx0_hbm, idx1_hbm, tab_hbm, out_hbm,
              idx0_v, idx1_v, buf, ga0, ga1, gb0, gb1):
    wid = lax.axis_index("s") * NC + lax.axis_index("c")
    gbase = wid * G
    pltpu.sync_copy(idx0_hbm.at[pl.ds(wid, 1)], idx0_v)
    pltpu.sync_copy(idx1_hbm.at[pl.ds(wid, 1)], idx1_v)

    ga = (ga0, ga1)
    gb = (gb0, gb1)

    def first_gather(j, slot, sem):
        return pltpu.async_copy(tab_hbm.at[idx0_v.at[0, j]], buf.at[slot], sem)

    def finish_group(j, slot):
        pltpu.make_async_copy(tab_hbm.at[idx0_v.at[0, j]], buf.at[slot],
                              ga[slot]).wait()
        pltpu.async_copy(tab_hbm.at[idx1_v.at[0, j]], buf.at[slot],
                         gb[slot], add=True).wait()
        pltpu.sync_copy(buf.at[slot], out_hbm.at[pl.ds((gbase + j) * CH, CH)])

    first_gather(0, 0, ga0)

    def step(it, carry):
        jj = it * 2
        first_gather(jj + 1, 1, ga1)
        finish_group(jj, 0)
        first_gather(jj + 2, 0, ga0)
        finish_group(jj + 1, 1)
        return carry

    lax.fori_loop(0, (G - 2) // 2, step, 0)

    first_gather(G - 1, 1, ga1)
    finish_group(G - 2, 0)
    finish_group(G - 1, 1)


def kernel(X, w_embed):
    Xf = X.reshape(N, 2).astype(jnp.int32)
    idx0 = Xf[:, 0].reshape(NW, G, CH)
    idx1 = Xf[:, 1].reshape(NW, G, CH)
    h = _sc_embed(idx0, idx1, w_embed)
    return h.reshape(B, T, D), w_embed


# speedup vs baseline: 6.9419x; 1.0819x over previous
import functools

import jax
import jax.numpy as jnp
from jax import lax
from jax.experimental import pallas as pl
from jax.experimental.pallas import tpu as pltpu
from jax.experimental.pallas import tpu_sc as plsc

B, T, D = 1024, 200, 64
N = B * T
NC, NS, L = 2, 16, 16
NW = NC * NS
PER_W = N // NW
CH = 128
G = PER_W // CH
HALVES = 2
R = CH * HALVES
GM = G // HALVES
NBUF = 3

_mesh = plsc.VectorSubcoreMesh(core_axis_name="c", subcore_axis_name="s")


@functools.partial(
    pl.kernel,
    mesh=_mesh,
    out_type=jax.ShapeDtypeStruct((N, D), jnp.float32),
    scratch_types=[
        pltpu.VMEM((1, G, CH), jnp.int32),
        pltpu.VMEM((1, G, CH), jnp.int32),
        pltpu.VMEM((NBUF, R, D), jnp.float32),
        [pltpu.SemaphoreType.DMA] * NBUF,
        [pltpu.SemaphoreType.DMA] * NBUF,
        [pltpu.SemaphoreType.DMA] * NBUF,
    ],
    compiler_params=pltpu.CompilerParams(use_tc_tiling_on_sc=False),
)
def _sc_embed(idx0_hbm, idx1_hbm, tab_hbm, out_hbm,
              idx0_v, idx1_v, buf, ga, gb, gc):
    wid = lax.axis_index("s") * NC + lax.axis_index("c")
    gbase = wid * G
    pltpu.sync_copy(idx0_hbm.at[pl.ds(wid, 1)], idx0_v)
    pltpu.sync_copy(idx1_hbm.at[pl.ds(wid, 1)], idx1_v)

    def gather(j, idx_v, sem, add):
        b = j % NBUF
        for h in range(HALVES):
            pltpu.async_copy(tab_hbm.at[idx_v.at[0, HALVES * j + h]],
                             buf.at[b, pl.ds(h * CH, CH)], sem[b], add=add)

    def wait_gather(j, idx_v, sem):
        b = j % NBUF
        for h in range(HALVES):
            pltpu.make_async_copy(tab_hbm.at[idx_v.at[0, HALVES * j + h]],
                                  buf.at[b, pl.ds(h * CH, CH)], sem[b]).wait()

    def scatter(j):
        b = j % NBUF
        pltpu.async_copy(buf.at[b], out_hbm.at[pl.ds((gbase + HALVES * j) * CH, R)],
                         gc[b])

    def wait_scatter(j):
        b = j % NBUF
        pltpu.make_async_copy(buf.at[b], out_hbm.at[pl.ds((gbase + HALVES * j) * CH, R)],
                              gc[b]).wait()

    gather(0, idx0_v, ga, False)
    gather(1, idx0_v, ga, False)
    for j in range(GM):
        wait_gather(j, idx0_v, ga)
        gather(j, idx1_v, gb, True)
        wait_gather(j, idx1_v, gb)
        scatter(j)
        if j + 2 < GM:
            if j >= 1:
                wait_scatter(j - 1)
            gather(j + 2, idx0_v, ga, False)
    for j in range(GM - 3, GM):
        wait_scatter(j)


def kernel(X, w_embed):
    Xf = X.reshape(N, 2).astype(jnp.int32)
    idx0 = Xf[:, 0].reshape(NW, G, CH)
    idx1 = Xf[:, 1].reshape(NW, G, CH)
    h = _sc_embed(idx0, idx1, w_embed)
    return h.reshape(B, T, D), w_embed


# speedup vs baseline: 6.9942x; 1.0075x over previous
import functools

import jax
import jax.numpy as jnp
from jax import lax
from jax.experimental import pallas as pl
from jax.experimental.pallas import tpu as pltpu
from jax.experimental.pallas import tpu_sc as plsc

B, T, D = 1024, 200, 64
N = B * T
NC, NS, L = 2, 16, 16
NW = NC * NS
PER_W = N // NW
CH = 128
G = PER_W // CH
HALVES = 2
R = CH * HALVES
GM = G // HALVES
NBUF = 5

_mesh = plsc.VectorSubcoreMesh(core_axis_name="c", subcore_axis_name="s")


@functools.partial(
    pl.kernel,
    mesh=_mesh,
    out_type=jax.ShapeDtypeStruct((N, D), jnp.float32),
    scratch_types=[
        pltpu.VMEM((1, G, CH), jnp.int32),
        pltpu.VMEM((1, G, CH), jnp.int32),
        pltpu.VMEM((NBUF, R, D), jnp.float32),
        [pltpu.SemaphoreType.DMA] * NBUF,
        [pltpu.SemaphoreType.DMA] * NBUF,
        [pltpu.SemaphoreType.DMA] * NBUF,
    ],
    compiler_params=pltpu.CompilerParams(use_tc_tiling_on_sc=False),
)
def _sc_embed(idx0_hbm, idx1_hbm, tab_hbm, out_hbm,
              idx0_v, idx1_v, buf, ga, gb, gc):
    wid = lax.axis_index("s") * NC + lax.axis_index("c")
    gbase = wid * G
    pltpu.sync_copy(idx0_hbm.at[pl.ds(wid, 1)], idx0_v)
    pltpu.sync_copy(idx1_hbm.at[pl.ds(wid, 1)], idx1_v)

    def gather(j, idx_v, sem, add):
        b = j % NBUF
        for h in range(HALVES):
            pltpu.async_copy(tab_hbm.at[idx_v.at[0, HALVES * j + h]],
                             buf.at[b, pl.ds(h * CH, CH)], sem[b], add=add)

    def wait_gather(j, idx_v, sem):
        b = j % NBUF
        for h in range(HALVES):
            pltpu.make_async_copy(tab_hbm.at[idx_v.at[0, HALVES * j + h]],
                                  buf.at[b, pl.ds(h * CH, CH)], sem[b]).wait()

    def scatter(j):
        b = j % NBUF
        pltpu.async_copy(buf.at[b], out_hbm.at[pl.ds((gbase + HALVES * j) * CH, R)],
                         gc[b])

    def wait_scatter(j):
        b = j % NBUF
        pltpu.make_async_copy(buf.at[b], out_hbm.at[pl.ds((gbase + HALVES * j) * CH, R)],
                              gc[b]).wait()

    for j in range(3):
        gather(j, idx0_v, ga, False)
    for j in range(GM):
        if j >= 2 and j + 3 < GM:
            wait_scatter(j - 2)
        if j + 3 < GM:
            gather(j + 3, idx0_v, ga, False)
        wait_gather(j, idx0_v, ga)
        gather(j, idx1_v, gb, True)
        if j >= 1:
            wait_gather(j - 1, idx1_v, gb)
            scatter(j - 1)
    wait_gather(GM - 1, idx1_v, gb)
    scatter(GM - 1)
    for j in range(GM - 5, GM):
        wait_scatter(j)


def kernel(X, w_embed):
    Xf = X.reshape(N, 2).astype(jnp.int32)
    idx0 = Xf[:, 0].reshape(NW, G, CH)
    idx1 = Xf[:, 1].reshape(NW, G, CH)
    h = _sc_embed(idx0, idx1, w_embed)
    return h.reshape(B, T, D), w_embed
